# Initial kernel scaffold; baseline (speedup 1.0000x reference)
import jax
import jax.numpy as jnp
from jax.experimental import pallas as pl


def kernel(feat, edge_index, edge_weight, ntype_idxs, etype_idxs, W1, a_l1, a_r1, W2, a_l2, a_r2, res_W2, res_b2):
    raise NotImplementedError("write your pallas kernel here")


# speedup vs baseline: 16.4570x; 16.4570x over previous
import functools

import jax
import jax.numpy as jnp
from jax import lax
from jax.experimental import pallas as pl
from jax.experimental.pallas import tpu as pltpu
from jax.experimental.pallas import tpu_sc as plsc

N = 10000
E = 160000
T = 4
D_IN = 128
HEADS1 = 8
HID = 16
OUT = 64
SLOPE = 0.2

NC = 2
NS = 16
NWK = NC * NS
CH = 128
EPW = 5120
NCHUNK = EPW // CH
EP = NWK * EPW
NP = 10240
ROWS_PER_TILE = NP // NS

D1 = 144
W1COLS = 128
D2 = 80
W2COLS = 64

BN = 1000


def _tables_body(x_ref, p_ref, z_ref):
    z_ref[0] = jnp.dot(x_ref[...], p_ref[0], preferred_element_type=jnp.float32)


def _build_tables(x, P):
    T_, K, Do = P.shape
    return pl.pallas_call(
        _tables_body,
        grid=(T_, N // BN),
        in_specs=[
            pl.BlockSpec((BN, K), lambda t, i: (i, 0)),
            pl.BlockSpec((1, K, Do), lambda t, i: (t, 0, 0)),
        ],
        out_specs=pl.BlockSpec((1, BN, Do), lambda t, i: (t, i, 0)),
        out_shape=jax.ShapeDtypeStruct((T_, N, Do), jnp.float32),
    )(x, P)


def _layer2_body(prt_ref, p2_ref, rw_ref, rb_ref, z_ref, r_ref, h_ref):
    t = pl.program_id(1)

    @pl.when(t == 0)
    def _():
        x = prt_ref[0] + prt_ref[1]
        h_ref[...] = jnp.where(x > 0, x, jnp.exp(x) - 1.0)
        r_ref[...] = (
            jnp.dot(h_ref[...], rw_ref[...], preferred_element_type=jnp.float32)
            + rb_ref[...]
        )

    z_ref[0] = jnp.dot(h_ref[...], p2_ref[0], preferred_element_type=jnp.float32)


def _layer2_tables(prt, P2, rw, rb):
    return pl.pallas_call(
        _layer2_body,
        grid=(N // BN, T),
        in_specs=[
            pl.BlockSpec((2, BN, D_IN), lambda i, t: (0, i, 0)),
            pl.BlockSpec((1, D_IN, D2), lambda i, t: (t, 0, 0)),
            pl.BlockSpec((D_IN, OUT), lambda i, t: (0, 0)),
            pl.BlockSpec((1, OUT), lambda i, t: (0, 0)),
        ],
        out_specs=[
            pl.BlockSpec((1, BN, D2), lambda i, t: (t, i, 0)),
            pl.BlockSpec((BN, OUT), lambda i, t: (i, 0)),
        ],
        out_shape=[
            jax.ShapeDtypeStruct((T, N, D2), jnp.float32),
            jax.ShapeDtypeStruct((N, OUT), jnp.float32),
        ],
        scratch_shapes=[pltpu.VMEM((BN, D_IN), jnp.float32)],
    )(prt, P2, rw, rb)


def _final_body(q_ref, r_ref, o_ref):
    o_ref[...] = q_ref[0] + q_ref[1] + r_ref[...]


def _final_combine(q, R):
    return pl.pallas_call(
        _final_body,
        grid=(N // BN,),
        in_specs=[
            pl.BlockSpec((2, BN, OUT), lambda i: (0, i, 0)),
            pl.BlockSpec((BN, OUT), lambda i: (i, 0)),
        ],
        out_specs=pl.BlockSpec((BN, OUT), lambda i: (i, 0)),
        out_shape=jax.ShapeDtypeStruct((N, OUT), jnp.float32),
    )(q, R)


def _make_sc_edge_kernel(D, W, NH):
    mesh = plsc.VectorSubcoreMesh(core_axis_name="c", subcore_axis_name="s")

    @functools.partial(
        pl.kernel,
        mesh=mesh,
        out_type=jax.ShapeDtypeStruct((NC * NP, W), jnp.float32),
        compiler_params=pltpu.CompilerParams(use_tc_tiling_on_sc=False),
        scratch_types=[
            pltpu.VMEM((CH,), jnp.int32),
            pltpu.VMEM((CH,), jnp.int32),
            pltpu.VMEM((CH,), jnp.float32),
            pltpu.VMEM((CH, D), jnp.float32),
            pltpu.VMEM((CH, W), jnp.float32),
            pltpu.VMEM_SHARED((NP, W), jnp.float32),
            pltpu.SemaphoreType.DMA,
        ],
    )
    def k(table_h, si_h, dst_h, ew_h, zero_h, out_h,
          si_v, dst_v, ew_v, rows_v, al_v, acc_sh, sem):
        cid = lax.axis_index("c")
        sid = lax.axis_index("s")
        wid = sid * NC + cid
        r0 = pl.multiple_of(sid * ROWS_PER_TILE, 8)
        pltpu.sync_copy(zero_h.at[pl.ds(r0, ROWS_PER_TILE)],
                        acc_sh.at[pl.ds(r0, ROWS_PER_TILE)])
        plsc.subcore_barrier()

        base0 = wid * EPW

        def chunk(c, carry):
            base = base0 + c * CH
            pltpu.sync_copy(si_h.at[pl.ds(base, CH)], si_v)
            pltpu.sync_copy(dst_h.at[pl.ds(base, CH)], dst_v)
            pltpu.sync_copy(ew_h.at[pl.ds(base, CH)], ew_v)
            pltpu.async_copy(table_h.at[si_v], rows_v, sem).wait()

            def group(g, carry2):
                ew16 = ew_v[pl.ds(g * 16, 16)]
                for j in range(16):
                    e = g * 16 + j
                    lv = rows_v[e, pl.ds(W, 16)]
                    lv = jnp.where(lv >= 0, lv, SLOPE * lv)
                    att = (1.0 / (1.0 + jnp.exp(-lv))) * ew16[j]
                    for v in range(W // 16):
                        hk = (v * NH * 16) // W
                        al_v[e, pl.ds(v * 16, 16)] = (
                            rows_v[e, pl.ds(v * 16, 16)] * att[hk]
                        )
                return carry2

            lax.fori_loop(0, CH // 16, group, 0)
            pltpu.sync_copy(al_v, acc_sh.at[dst_v], add=True)
            return carry

        lax.fori_loop(0, NCHUNK, chunk, 0)
        plsc.subcore_barrier()
        pltpu.sync_copy(acc_sh.at[pl.ds(r0, ROWS_PER_TILE)],
                        out_h.at[pl.ds(cid * NP + r0, ROWS_PER_TILE)])

    return k


_sc_layer1 = _make_sc_edge_kernel(D1, W1COLS, HEADS1)
_sc_layer2 = _make_sc_edge_kernel(D2, W2COLS, 1)


def kernel(feat, edge_index, edge_weight, ntype_idxs, etype_idxs,
           W1, a_l1, a_r1, W2, a_l2, a_r2, res_W2, res_b2):
    src = edge_index[0]
    dst = edge_index[1]

    a1 = (a_l1 + a_r1).reshape(T, D_IN, HEADS1, HID).sum(-1)
    P1 = jnp.concatenate(
        [W1, jnp.matmul(W1, a1), jnp.zeros((T, D_IN, 8), jnp.float32)], axis=2)

    idxc = jnp.arange(D_IN)
    perm = (idxc % HID) * HEADS1 + idxc // HID
    W2p = W2[:, perm, :]
    rwp = res_W2[perm, :]
    a2 = (a_l2 + a_r2).sum(axis=2)
    C2 = jnp.einsum('tko,to->tk', W2p, a2)
    P2 = jnp.concatenate(
        [W2p, C2[:, :, None], jnp.zeros((T, D_IN, D2 - OUT - 1), jnp.float32)],
        axis=2)

    pad = EP - E
    si = jnp.concatenate([etype_idxs * N + src,
                          jnp.zeros((pad,), jnp.int32)])
    dstp = jnp.concatenate([dst, jnp.zeros((pad,), jnp.int32)])
    ewp = jnp.concatenate([edge_weight, jnp.zeros((pad,), jnp.float32)])

    Z1 = _build_tables(feat, P1).reshape(T * N, D1)
    zeros1 = jnp.zeros((NP, W1COLS), jnp.float32)
    prt = _sc_layer1(Z1, si, dstp, ewp, zeros1).reshape(NC, NP, D_IN)[:, :N]

    Z2_R = _layer2_tables(prt, P2, rwp, res_b2.reshape(1, OUT))
    Z2 = Z2_R[0].reshape(T * N, D2)
    R = Z2_R[1]

    zeros2 = jnp.zeros((NP, W2COLS), jnp.float32)
    q = _sc_layer2(Z2, si, dstp, ewp, zeros2).reshape(NC, NP, OUT)[:, :N]
    return _final_combine(q, R)
